# baseline (speedup 1.0000x reference)
import functools

import jax
import jax.numpy as jnp
from jax import lax
from jax.experimental import pallas as pl
from jax.experimental.pallas import tpu as pltpu
from jax.experimental.pallas import tpu_sc as plsc

_USR_DIM = 128
_PRD_DIM = 64
_B = 4096
_NSLOT = 10


@functools.cache
def _build():
    info = plsc.get_sparse_core_info()
    nc, ns = info.num_cores, info.num_subcores
    nw = nc * ns
    bpw = _B // nw

    mesh = plsc.VectorSubcoreMesh(core_axis_name="c", subcore_axis_name="s")

    @functools.partial(
        pl.kernel,
        mesh=mesh,
        out_type=[
            jax.ShapeDtypeStruct((_B, _USR_DIM), jnp.float32),
            jax.ShapeDtypeStruct((_PRD_DIM, _B), jnp.float32),
        ],
        scratch_types=[
            pltpu.VMEM((bpw,), jnp.int32),
            pltpu.VMEM((bpw, _USR_DIM), jnp.float32),
            pltpu.VMEM((bpw,), jnp.int32),
            [pltpu.VMEM((_PRD_DIM, 128), jnp.float32) for _ in range(_NSLOT)],
            pltpu.VMEM((_PRD_DIM, bpw), jnp.float32),
            pltpu.SemaphoreType.DMA,
            [pltpu.SemaphoreType.DMA for _ in range(_NSLOT)],
        ],
        compiler_params=pltpu.CompilerParams(disable_bounds_checks=True,
                                             needs_layout_passes=False),
    )
    def gather_kernel(uids_hbm, iids_hbm, utab_hbm, itabT_hbm,
                      uout_hbm, ioutT_hbm,
                      uidx_v, urows_v, iidx_v, blks, ioutT_v, usem, isems):
        wid = lax.axis_index("s") * nc + lax.axis_index("c")
        base = wid * bpw
        pltpu.sync_copy(uids_hbm.at[pl.ds(base, bpw)], uidx_v)
        pltpu.sync_copy(iids_hbm.at[pl.ds(base, bpw)], iidx_v)
        ucp = pltpu.async_copy(utab_hbm.at[uidx_v], urows_v, usem)

        iota = lax.iota(jnp.int32, 16)
        rowv = [iota + 16 * k for k in range(_PRD_DIM // 16)]

        def issue(v, s):
            col0 = pl.multiple_of(v - (v & 127), 128)
            pltpu.async_copy(itabT_hbm.at[:, pl.ds(col0, 128)],
                             blks[s], isems[s])

        def select(v, s, r):
            pltpu.make_async_copy(itabT_hbm.at[:, pl.ds(0, 128)],
                                  blks[s], isems[s]).wait()
            colv = jnp.full((16,), v & 127, jnp.int32)
            rv = jnp.full((16,), r, jnp.int32)
            for k in range(_PRD_DIM // 16):
                vals = plsc.load_gather(blks[s], [rowv[k], colv])
                plsc.store_scatter(ioutT_v, [rowv[k], rv], vals)

        for j in range(bpw // 16):
            v = iidx_v[pl.ds(j * 16, 16)]
            if j == 0:
                for l in range(_NSLOT):
                    issue(v[l], l)
            for l in range(16):
                r = j * 16 + l
                nxt = r + _NSLOT
                select(v[l], r % _NSLOT, r)
                if nxt < bpw:
                    jn, ln = divmod(nxt, 16)
                    if jn == j:
                        issue(v[ln], nxt % _NSLOT)
                    else:
                        vn = iidx_v[pl.ds(jn * 16, 16)]
                        issue(vn[ln], nxt % _NSLOT)

        ucp.wait()
        pltpu.sync_copy(urows_v, uout_hbm.at[pl.ds(base, bpw)])
        pltpu.sync_copy(ioutT_v, ioutT_hbm.at[:, pl.ds(base, bpw)])

    return gather_kernel


def kernel(user_ids, item_ids, user_table, item_table):
    fn = _build()
    u, ioT = fn(user_ids.astype(jnp.int32), item_ids.astype(jnp.int32),
                user_table, item_table.T)
    return u[:, None, :], ioT.T[:, None, :]


# speedup vs baseline: 1.0677x; 1.0677x over previous
import functools

import jax
import jax.numpy as jnp
from jax import lax
from jax.experimental import pallas as pl
from jax.experimental.pallas import tpu as pltpu
from jax.experimental.pallas import tpu_sc as plsc

_USR_DIM = 128
_PRD_DIM = 64
_B = 4096
_NSLOT = 8


@functools.cache
def _build():
    info = plsc.get_sparse_core_info()
    nc, ns = info.num_cores, info.num_subcores
    nw = nc * ns
    bpw = _B // nw

    mesh = plsc.VectorSubcoreMesh(core_axis_name="c", subcore_axis_name="s")

    @functools.partial(
        pl.kernel,
        mesh=mesh,
        out_type=[
            jax.ShapeDtypeStruct((_B, _USR_DIM), jnp.float32),
            jax.ShapeDtypeStruct((_PRD_DIM, _B), jnp.float32),
        ],
        scratch_types=[
            pltpu.VMEM((bpw,), jnp.int32),
            pltpu.VMEM((bpw, _USR_DIM), jnp.float32),
            pltpu.VMEM((bpw,), jnp.int32),
            [pltpu.VMEM((_PRD_DIM, 128), jnp.float32) for _ in range(_NSLOT)],
            pltpu.VMEM((_PRD_DIM, bpw), jnp.float32),
            pltpu.SemaphoreType.DMA,
            [pltpu.SemaphoreType.DMA for _ in range(_NSLOT)],
        ],
        compiler_params=pltpu.CompilerParams(disable_bounds_checks=True,
                                             needs_layout_passes=False),
    )
    def gather_kernel(uids_hbm, iids_hbm, utab_hbm, itabT_hbm,
                      uout_hbm, ioutT_hbm,
                      uidx_v, urows_v, iidx_v, blks, ioutT_v, usem, isems):
        wid = lax.axis_index("s") * nc + lax.axis_index("c")
        base = wid * bpw
        pltpu.sync_copy(uids_hbm.at[pl.ds(base, bpw)], uidx_v)
        pltpu.sync_copy(iids_hbm.at[pl.ds(base, bpw)], iidx_v)
        ucp = pltpu.async_copy(utab_hbm.at[uidx_v], urows_v, usem)

        iota = lax.iota(jnp.int32, 16)
        rowv = [iota + 16 * k for k in range(_PRD_DIM // 16)]

        def issue(v, s):
            col0 = pl.multiple_of(v - (v & 127), 128)
            pltpu.async_copy(itabT_hbm.at[:, pl.ds(col0, 128)],
                             blks[s], isems[s])

        def select(v, s, r):
            pltpu.make_async_copy(itabT_hbm.at[:, pl.ds(0, 128)],
                                  blks[s], isems[s]).wait()
            colv = jnp.full((16,), v & 127, jnp.int32)
            rv = jnp.full((16,), r, jnp.int32)
            for k in range(_PRD_DIM // 16):
                vals = plsc.load_gather(blks[s], [rowv[k], colv])
                plsc.store_scatter(ioutT_v, [rowv[k], rv], vals)

        v0 = iidx_v[pl.ds(0, 16)]
        for l in range(_NSLOT):
            issue(v0[l], l)

        def chunk_body(j, carry):
            v = iidx_v[pl.ds(j * 16, 16)]
            r0 = j * 16
            for l in range(8):
                select(v[l], l, r0 + l)
                issue(v[l + 8], l)

            @pl.when(j < bpw // 16 - 1)
            def _():
                vn = iidx_v[pl.ds((j + 1) * 16, 16)]
                for l in range(8):
                    select(v[l + 8], l, r0 + l + 8)
                    issue(vn[l], l)

            @pl.when(j == bpw // 16 - 1)
            def _():
                for l in range(8):
                    select(v[l + 8], l, r0 + l + 8)

            return carry

        lax.fori_loop(0, bpw // 16, chunk_body, 0)

        ucp.wait()
        pltpu.sync_copy(urows_v, uout_hbm.at[pl.ds(base, bpw)])
        pltpu.sync_copy(ioutT_v, ioutT_hbm.at[:, pl.ds(base, bpw)])

    return gather_kernel


def kernel(user_ids, item_ids, user_table, item_table):
    fn = _build()
    u, ioT = fn(user_ids.astype(jnp.int32), item_ids.astype(jnp.int32),
                user_table, item_table.T)
    return u[:, None, :], ioT.T[:, None, :]
